# baseline (speedup 1.0000x reference)
import functools

import jax
import jax.numpy as jnp
from jax import lax
from jax.experimental import pallas as pl
from jax.experimental.pallas import tpu as pltpu
from jax.experimental.pallas import tpu_sc as plsc

_EPS = 1e-06
_HB = 128


def _ln_body(bits_ref, x_ref, w_ref, b_ref, o_ref):
    bidx = pl.program_id(0)
    hb = pl.program_id(1)
    x = x_ref[...]
    u = jnp.mean(x, axis=2, keepdims=True)
    s2 = jnp.mean(x * x, axis=2, keepdims=True)
    v = s2 - u * u
    r = lax.rsqrt(v + _EPS)
    ln = (x - u) * (r * w_ref[...]) + b_ref[...]
    wc = jnp.right_shift(lax.broadcasted_iota(jnp.int32, (1, 1, 1, 128), 3), 3)
    for k in range(_HB // 8):
        s = bits_ref[bidx, hb * (_HB // 8) + k]
        bit = jnp.right_shift(s, wc) & 1
        o_ref[0, 8 * k:8 * k + 8] = jnp.where(
            bit[0] != 0, ln[0, 8 * k:8 * k + 8], 0.0)


def _pack_bits_sc(act_flat):
    B = act_flat.shape[0]
    info = plsc.get_sparse_core_info()
    nc = info.num_cores
    mesh = plsc.VectorSubcoreMesh(core_axis_name="c", subcore_axis_name="s")

    @functools.partial(
        pl.kernel, mesh=mesh,
        out_type=jax.ShapeDtypeStruct((B, 16), jnp.int32),
        scratch_types=[
            pltpu.VMEM((256,), jnp.int32),
            pltpu.VMEM((16,), jnp.int32),
        ],
    )
    def k(act_hbm, out_hbm, act_v, out_v):
        wid = lax.axis_index("s") * nc + lax.axis_index("c")
        pltpu.sync_copy(act_hbm.at[wid], act_v)
        acc = jnp.zeros((16,), jnp.int32)
        for wcell in range(16):
            acc = acc | (act_v[pl.ds(16 * wcell, 16)] << wcell)
        out_v[...] = acc
        pltpu.sync_copy(out_v, out_hbm.at[wid])

    return k(act_flat)


def kernel(x, active, weight, bias):
    B, H, W, C = x.shape
    bits = _pack_bits_sc(active[:, 0].transpose(0, 2, 1).reshape(B, 256))
    w_bc = jnp.broadcast_to(weight[:, None], (C, W))
    b_bc = jnp.broadcast_to(bias[:, None], (C, W))
    xt = jnp.transpose(x, (0, 1, 3, 2))
    out_t = pl.pallas_call(
        _ln_body,
        grid=(B, H // _HB),
        in_specs=[
            pl.BlockSpec(memory_space=pltpu.SMEM),
            pl.BlockSpec((1, _HB, C, W), lambda b, h: (b, h, 0, 0)),
            pl.BlockSpec((C, W), lambda b, h: (0, 0)),
            pl.BlockSpec((C, W), lambda b, h: (0, 0)),
        ],
        out_specs=pl.BlockSpec((1, _HB, C, W), lambda b, h: (b, h, 0, 0)),
        out_shape=jax.ShapeDtypeStruct((B, H, C, W), x.dtype),
        compiler_params=pltpu.CompilerParams(
            dimension_semantics=("parallel", "parallel")),
    )(bits, xt, w_bc, b_bc)
    return jnp.transpose(out_t, (0, 1, 3, 2))


# speedup vs baseline: 1.1359x; 1.1359x over previous
import jax
import jax.numpy as jnp
from jax import lax
from jax.experimental import pallas as pl
from jax.experimental.pallas import tpu as pltpu

_EPS = 1e-06
_HB = 128


def _ln_body(bits_ref, x_ref, w_ref, b_ref, o_ref):
    bidx = pl.program_id(0)
    hb = pl.program_id(1)
    x = x_ref[...]
    u = jnp.mean(x, axis=2, keepdims=True)
    s2 = jnp.mean(x * x, axis=2, keepdims=True)
    v = s2 - u * u
    r = lax.rsqrt(v + _EPS)
    ln = (x - u) * (r * w_ref[...]) + b_ref[...]
    wc = jnp.right_shift(lax.broadcasted_iota(jnp.int32, (1, 1, 1, 128), 3), 3)
    for k in range(_HB // 8):
        s = bits_ref[bidx, hb * (_HB // 8) + k]
        bit = jnp.right_shift(s, wc) & 1
        o_ref[0, 8 * k:8 * k + 8] = jnp.where(
            bit[0] != 0, ln[0, 8 * k:8 * k + 8], 0.0)


def kernel(x, active, weight, bias):
    B, H, W, C = x.shape
    bits = jnp.sum(active[:, 0].astype(jnp.int32) << jnp.arange(16, dtype=jnp.int32),
                   axis=-1, dtype=jnp.int32)
    w_bc = jnp.broadcast_to(weight[:, None], (C, W))
    b_bc = jnp.broadcast_to(bias[:, None], (C, W))
    xt = jnp.transpose(x, (0, 1, 3, 2))
    out_t = pl.pallas_call(
        _ln_body,
        grid=(B, H // _HB),
        in_specs=[
            pl.BlockSpec(memory_space=pltpu.SMEM),
            pl.BlockSpec((1, _HB, C, W), lambda b, h: (b, h, 0, 0)),
            pl.BlockSpec((C, W), lambda b, h: (0, 0)),
            pl.BlockSpec((C, W), lambda b, h: (0, 0)),
        ],
        out_specs=pl.BlockSpec((1, _HB, C, W), lambda b, h: (b, h, 0, 0)),
        out_shape=jax.ShapeDtypeStruct((B, H, C, W), x.dtype),
        compiler_params=pltpu.CompilerParams(
            dimension_semantics=("parallel", "parallel")),
    )(bits, xt, w_bc, b_bc)
    return jnp.transpose(out_t, (0, 1, 3, 2))
